# Initial kernel scaffold; baseline (speedup 1.0000x reference)
import jax
import jax.numpy as jnp
from jax.experimental import pallas as pl


def kernel(x, edge_index, W_conv, W_proj):
    raise NotImplementedError("write your pallas kernel here")


# speedup vs baseline: 4.8385x; 4.8385x over previous
import jax
import jax.numpy as jnp
from jax import lax
from jax.experimental import pallas as pl
from jax.experimental.pallas import tpu as pltpu
from jax.experimental.pallas import tpu_sc as plsc

N_NODES = 10000
N_EDGES = 320000
D_IN = 128
D_OUT = 256

NC = 2
NS = 16
NW = NC * NS
E_PER_W = N_EDGES // NW
CHUNK = 80
N_CHUNKS = E_PER_W // CHUNK
N_PAD = 10240
ROWS_PER_TILE = N_PAD // NS
ZROWS = 128


DEG_W = 128


def _fill_16lane(ref, nrow, ncol, value):
    nv = ncol // 16
    def row(r, _):
        def col(j, _):
            ref[r, pl.ds(j * 16, 16)] = jnp.full((16,), value, jnp.float32)
            return 0
        return lax.fori_loop(0, nv, col, 0)
    lax.fori_loop(0, nrow, row, 0)


def _sc_body(src_hbm, dst_hbm, x_hbm, agg_out,
             sidx_v, rows_v, zagg_v, agg_sh, sem):
    c = lax.axis_index("c")
    s = lax.axis_index("s")
    wid = c * NS + s

    _fill_16lane(zagg_v, ZROWS, D_IN, 0.0)

    base_row = s * ROWS_PER_TILE
    for k in range(ROWS_PER_TILE // ZROWS):
        pltpu.sync_copy(zagg_v, agg_sh.at[pl.ds(base_row + k * ZROWS, ZROWS)])
    plsc.subcore_barrier()

    def chunk(i, _):
        base = wid * E_PER_W + i * CHUNK
        pltpu.sync_copy(src_hbm.at[pl.ds(base, CHUNK)], sidx_v)
        pltpu.async_copy(x_hbm.at[sidx_v], rows_v, sem).wait()
        pltpu.sync_copy(dst_hbm.at[pl.ds(base, CHUNK)], sidx_v)
        pltpu.sync_copy(rows_v, agg_sh.at[sidx_v], add=True)
        return 0

    lax.fori_loop(0, N_CHUNKS, chunk, 0)
    plsc.subcore_barrier()

    pltpu.sync_copy(agg_sh.at[pl.ds(base_row, ROWS_PER_TILE)],
                    agg_out.at[c, pl.ds(base_row, ROWS_PER_TILE)])


def _sc_aggregate(src, dst, x):
    mesh = plsc.VectorSubcoreMesh(core_axis_name="c", subcore_axis_name="s")
    return pl.kernel(
        _sc_body,
        out_type=jax.ShapeDtypeStruct((NC, N_PAD, D_IN), jnp.float32),
        mesh=mesh,
        scratch_types=[
            pltpu.VMEM((CHUNK,), jnp.int32),
            pltpu.VMEM((CHUNK, D_IN), jnp.float32),
            pltpu.VMEM((ZROWS, D_IN), jnp.float32),
            pltpu.VMEM_SHARED((N_PAD, D_IN), jnp.float32),
            pltpu.SemaphoreType.DMA,
        ],
        name="sc_edge_aggregate",
    )(src, dst, x)


def _sc_deg_body(dst_hbm, deg_out, didx_v, ones_v, zdeg_v, deg_sh):
    c = lax.axis_index("c")
    s = lax.axis_index("s")
    wid = c * NS + s

    _fill_16lane(ones_v, CHUNK, DEG_W, 1.0)
    _fill_16lane(zdeg_v, ZROWS, DEG_W, 0.0)

    base_row = s * ROWS_PER_TILE
    for k in range(ROWS_PER_TILE // ZROWS):
        pltpu.sync_copy(zdeg_v, deg_sh.at[pl.ds(base_row + k * ZROWS, ZROWS)])
    plsc.subcore_barrier()

    def chunk(i, _):
        base = wid * E_PER_W + i * CHUNK
        pltpu.sync_copy(dst_hbm.at[pl.ds(base, CHUNK)], didx_v)
        pltpu.sync_copy(ones_v, deg_sh.at[didx_v], add=True)
        return 0

    lax.fori_loop(0, N_CHUNKS, chunk, 0)
    plsc.subcore_barrier()

    pltpu.sync_copy(deg_sh.at[pl.ds(base_row, ROWS_PER_TILE)],
                    deg_out.at[c, pl.ds(base_row, ROWS_PER_TILE)])


def _sc_degree(dst):
    mesh = plsc.VectorSubcoreMesh(core_axis_name="c", subcore_axis_name="s")
    return pl.kernel(
        _sc_deg_body,
        out_type=jax.ShapeDtypeStruct((NC, N_PAD, DEG_W), jnp.float32),
        mesh=mesh,
        scratch_types=[
            pltpu.VMEM((CHUNK,), jnp.int32),
            pltpu.VMEM((CHUNK, DEG_W), jnp.float32),
            pltpu.VMEM((ZROWS, DEG_W), jnp.float32),
            pltpu.VMEM_SHARED((N_PAD, DEG_W), jnp.float32),
        ],
        name="sc_degree",
    )(dst)


def _tc_body(x_ref, agg_ref, deg_ref, wc_ref, wp_ref, out_ref):
    aggs = agg_ref[0] + agg_ref[1]
    deg = deg_ref[0] + deg_ref[1]
    inv = 1.0 / jnp.maximum(deg, 1.0)
    h = jnp.dot(aggs * inv, wc_ref[...],
                preferred_element_type=jnp.float32)
    out_ref[...] = h + jnp.dot(x_ref[...], wp_ref[...],
                               preferred_element_type=jnp.float32)


def _tc_combine(x, agg_part, deg_part, W_conv, W_proj):
    blk = 1024
    grid = (N_PAD // blk,)
    return pl.pallas_call(
        _tc_body,
        grid=grid,
        in_specs=[
            pl.BlockSpec((blk, D_IN), lambda i: (i, 0)),
            pl.BlockSpec((NC, blk, D_IN), lambda i: (0, i, 0)),
            pl.BlockSpec((NC, blk, DEG_W), lambda i: (0, i, 0)),
            pl.BlockSpec((D_IN, D_OUT), lambda i: (0, 0)),
            pl.BlockSpec((D_IN, D_OUT), lambda i: (0, 0)),
        ],
        out_specs=pl.BlockSpec((blk, D_OUT), lambda i: (i, 0)),
        out_shape=jax.ShapeDtypeStruct((N_NODES, D_OUT), jnp.float32),
    )(x, agg_part, deg_part, W_conv, W_proj)


@jax.jit
def kernel(x, edge_index, W_conv, W_proj):
    src = edge_index[0]
    dst = edge_index[1]
    agg_part = _sc_aggregate(src, dst, x)
    deg_part = _sc_degree(dst)
    return _tc_combine(x, agg_part, deg_part, W_conv, W_proj)
